# baseline (speedup 1.0000x reference)
import functools

import jax
import jax.numpy as jnp
from jax import lax
from jax.experimental import pallas as pl
from jax.experimental.pallas import tpu as pltpu
from jax.experimental.pallas import tpu_sc as plsc

VOCAB = 1000
VPAD = 1024
B, T = 1024, 50
N = B * T
NC, NS = 2, 16
NW = NC * NS
NTR = VOCAB // 8
NPT = N // 128
WPT = 13
WCOLS = WPT * 128
PER_W = N // NW
NG = PER_W // 16


def _lse_body(table_ref, lse_ref):
    x = table_ref[...]
    m = jnp.max(x, axis=1, keepdims=True)
    s = jnp.sum(jnp.exp(x - m), axis=1, keepdims=True)
    lse_ref[...] = m + jnp.log(s)


def _row_lse(table):
    return pl.pallas_call(
        _lse_body,
        out_shape=jax.ShapeDtypeStruct((VOCAB, 1), jnp.float32),
    )(table)


def _fin_body(p_ref, loss_ref):
    loss_ref[...] = jnp.sum(p_ref[...], keepdims=True) * (1.0 / N)


def _finalize(partials):
    return pl.pallas_call(
        _fin_body,
        out_shape=jax.ShapeDtypeStruct((1, 1), jnp.float32),
    )(partials)


_mesh = plsc.VectorSubcoreMesh(core_axis_name="c", subcore_axis_name="s")


@functools.partial(
    pl.kernel,
    out_type=(
        jax.ShapeDtypeStruct((VOCAB, N), jnp.float32),
        jax.ShapeDtypeStruct((NW, 16), jnp.float32),
    ),
    mesh=_mesh,
    compiler_params=pltpu.CompilerParams(
        needs_layout_passes=False, use_tc_tiling_on_sc=True),
    scratch_types=(
        pltpu.VMEM((WPT, 128), jnp.int32),
        pltpu.VMEM((NG, 16), jnp.int32),
        pltpu.VMEM((NG, 16), jnp.int32),
        pltpu.VMEM((VOCAB,), jnp.float32),
        pltpu.VMEM((2, 8, VOCAB), jnp.float32),
        pltpu.VMEM((2, 8, WCOLS), jnp.float32),
        pltpu.VMEM((2, 16, 128), jnp.float32),
        pltpu.VMEM((16,), jnp.float32),
        pltpu.SemaphoreType.DMA,
        pltpu.SemaphoreType.DMA,
        pltpu.SemaphoreType.DMA,
        pltpu.SemaphoreType.DMA,
        pltpu.SemaphoreType.DMA,
        pltpu.SemaphoreType.DMA,
    ),
)
def _sc_gather(tablet, table8, idxw3, idxn3, tgtn3, lse, out, partials,
               idx_v, idxn_v, tgtn_v, lse_v, slab2, blk2, vrow2, acc_v,
               ssem0, ssem1, osem0, osem1, nsem0, nsem1):
    wid = lax.axis_index("s") * NC + lax.axis_index("c")
    ct_lo = (25 * wid) // 2
    col0 = ct_lo * 128
    pltpu.sync_copy(idxw3.at[wid], idx_v)
    pltpu.sync_copy(idxn3.at[wid], idxn_v)
    pltpu.sync_copy(tgtn3.at[wid], tgtn_v)
    pltpu.sync_copy(lse, lse_v)
    acc_v[...] = jnp.zeros((16,), jnp.float32)

    iota16 = jnp.arange(16, dtype=jnp.int32)

    def slab_load(tr, b):
        sem = ssem0 if b == 0 else ssem1
        pltpu.async_copy(tablet.at[pl.ds(8 * tr, 8)], slab2.at[b], sem)

    def slab_wait(b):
        sem = ssem0 if b == 0 else ssem1
        pltpu.make_async_copy(tablet.at[pl.ds(0, 8)], slab2.at[b],
                              sem).wait()

    def blk_out(tr, b):
        sem = osem0 if b == 0 else osem1
        pltpu.async_copy(blk2.at[b],
                         out.at[pl.ds(8 * tr, 8), pl.ds(col0, WCOLS)], sem)

    def blk_wait(b):
        sem = osem0 if b == 0 else osem1
        pltpu.make_async_copy(blk2.at[b],
                              out.at[pl.ds(0, 8), pl.ds(col0, WCOLS)],
                              sem).wait()

    def nll_issue(g, b):
        sem = nsem0 if b == 0 else nsem1
        iv = idxn_v[g, :]
        tg = tgtn_v[g, :]
        fi = iv * 8 + lax.shift_right_logical(tg, 7)
        pltpu.async_copy(table8.at[fi], vrow2.at[b], sem)

    def nll_wait(b):
        sem = nsem0 if b == 0 else nsem1
        pltpu.make_async_copy(table8.at[pl.ds(0, 16)], vrow2.at[b],
                              sem).wait()

    def nll_compute(g, b):
        nll_wait(b)
        iv = idxn_v[g, :]
        tg = tgtn_v[g, :]
        lane = lax.bitwise_and(tg, jnp.int32(127))
        val = plsc.load_gather(vrow2.at[b], [iota16, lane])
        lsev = plsc.load_gather(lse_v, [iv])
        acc_v[...] = acc_v[...] + (lsev - val)

    def fill_block(b):
        slab = slab2.at[b]
        blk = blk2.at[b]

        def tile_body(j, _):
            for s in range(8):
                srow = jnp.full((16,), s, dtype=jnp.int32)
                for g in range(8):
                    cols = idx_v[j, pl.ds(16 * g, 16)]
                    vals = plsc.load_gather(slab, [srow, cols])
                    blk[s, pl.ds(128 * j + 16 * g, 16)] = vals
            return 0

        lax.fori_loop(0, WPT, tile_body, 0)

    slab_load(0, 0)
    nll_issue(0, 0)

    def pair_body(i, carry):
        tr0 = 2 * i
        tr1 = tr0 + 1

        slab_wait(0)
        slab_load(tr1, 1)

        @pl.when(i > 0)
        def _():
            blk_wait(0)

        @pl.when(i < NG // 2)
        def _():
            nll_issue(2 * i + 1, 1)
            nll_compute(2 * i, 0)

        fill_block(0)
        blk_out(tr0, 0)

        slab_wait(1)
        slab_load(tr1 + 1, 0)

        @pl.when(i > 0)
        def _():
            blk_wait(1)

        @pl.when(i < NG // 2 - 1)
        def _():
            nll_issue(2 * i + 2, 0)

        @pl.when(i < NG // 2)
        def _():
            nll_compute(2 * i + 1, 1)

        fill_block(1)
        blk_out(tr1, 1)
        return carry

    lax.fori_loop(0, (NTR - 1) // 2, pair_body, 0)

    slab_wait(0)
    blk_wait(0)
    fill_block(0)
    blk_out(NTR - 1, 0)
    blk_wait(1)
    blk_wait(0)

    pltpu.sync_copy(acc_v, partials.at[wid])


def kernel(idx, targets, table):
    idxf = idx.reshape(N)
    idx2 = idxf.reshape(NPT, 128)
    ct_starts = [(25 * w) // 2 for w in range(NW)]
    idxw3 = jnp.stack([idx2[s:s + WPT] for s in ct_starts])
    idxn3 = idxf.reshape(NW, NG, 16)
    tgtn3 = targets.reshape(NW, NG, 16)
    lse = _row_lse(table).reshape(VOCAB)
    tablet = table.T
    table_p = jnp.pad(table, ((0, 0), (0, VPAD - VOCAB)))
    table8 = table_p.reshape(VOCAB * 8, 128)
    out_t, partials = _sc_gather(tablet, table8, idxw3, idxn3, tgtn3, lse)
    loss = _finalize(partials)[0, 0]
    return (out_t.T, loss)


# speedup vs baseline: 2.2445x; 2.2445x over previous
import functools

import jax
import jax.numpy as jnp
from jax import lax
from jax.experimental import pallas as pl
from jax.experimental.pallas import tpu as pltpu
from jax.experimental.pallas import tpu_sc as plsc

VOCAB = 1000
VPAD = 1024
B, T = 1024, 50
N = B * T
NC, NS = 2, 16
NW = NC * NS
NTR = VOCAB // 8
NPT = N // 128
WPT = 13
WCOLS = WPT * 128
PER_W = N // NW
NG = PER_W // 16


def _lse_body(table_ref, lse_ref):
    x = table_ref[...]
    m = jnp.max(x, axis=1, keepdims=True)
    s = jnp.sum(jnp.exp(x - m), axis=1, keepdims=True)
    lse_ref[...] = m + jnp.log(s)


def _row_lse(table):
    return pl.pallas_call(
        _lse_body,
        out_shape=jax.ShapeDtypeStruct((VOCAB, 1), jnp.float32),
    )(table)


def _fin_body(p_ref, loss_ref):
    loss_ref[...] = jnp.sum(p_ref[...], keepdims=True) * (1.0 / N)


def _finalize(partials):
    return pl.pallas_call(
        _fin_body,
        out_shape=jax.ShapeDtypeStruct((1, 1), jnp.float32),
    )(partials)


_mesh = plsc.VectorSubcoreMesh(core_axis_name="c", subcore_axis_name="s")


@functools.partial(
    pl.kernel,
    out_type=(
        jax.ShapeDtypeStruct((VOCAB, N), jnp.float32),
        jax.ShapeDtypeStruct((NW, 16), jnp.float32),
    ),
    mesh=_mesh,
    compiler_params=pltpu.CompilerParams(
        needs_layout_passes=False, use_tc_tiling_on_sc=True),
    scratch_types=(
        pltpu.VMEM((WPT, 128), jnp.int32),
        pltpu.VMEM((NG, 16), jnp.int32),
        pltpu.VMEM((NG, 16), jnp.int32),
        pltpu.VMEM((VOCAB,), jnp.float32),
        pltpu.VMEM((8192,), jnp.float32),
        pltpu.VMEM((8192,), jnp.float32),
        pltpu.VMEM((2, 8, WCOLS), jnp.float32),
        pltpu.VMEM((2, 16, 128), jnp.float32),
        pltpu.VMEM((16,), jnp.float32),
        pltpu.SemaphoreType.DMA,
        pltpu.SemaphoreType.DMA,
        pltpu.SemaphoreType.DMA,
        pltpu.SemaphoreType.DMA,
        pltpu.SemaphoreType.DMA,
        pltpu.SemaphoreType.DMA,
    ),
)
def _sc_gather(tabletf, table8, idxw3, idxn3, tgtn3, lse, out, partials,
               idx_v, idxn_v, tgtn_v, lse_v, slab0, slab1, blk2, vrow2,
               acc_v, ssem0, ssem1, osem0, osem1, nsem0, nsem1):
    wid = lax.axis_index("s") * NC + lax.axis_index("c")
    ct_lo = (25 * wid) // 2
    col0 = ct_lo * 128
    pltpu.sync_copy(idxw3.at[wid], idx_v)
    pltpu.sync_copy(idxn3.at[wid], idxn_v)
    pltpu.sync_copy(tgtn3.at[wid], tgtn_v)
    pltpu.sync_copy(lse, lse_v)
    acc_v[...] = jnp.zeros((16,), jnp.float32)

    iota16 = jnp.arange(16, dtype=jnp.int32)

    def slab_load(tr, b):
        sem = ssem0 if b == 0 else ssem1
        slab = slab0 if b == 0 else slab1
        pltpu.async_copy(tabletf.at[pl.ds(8192 * tr, 8192)], slab, sem)

    def slab_wait(b):
        sem = ssem0 if b == 0 else ssem1
        slab = slab0 if b == 0 else slab1
        pltpu.make_async_copy(tabletf.at[pl.ds(0, 8192)], slab, sem).wait()

    def blk_out(tr, b):
        sem = osem0 if b == 0 else osem1
        pltpu.async_copy(blk2.at[b],
                         out.at[pl.ds(8 * tr, 8), pl.ds(col0, WCOLS)], sem)

    def blk_wait(b):
        sem = osem0 if b == 0 else osem1
        pltpu.make_async_copy(blk2.at[b],
                              out.at[pl.ds(0, 8), pl.ds(col0, WCOLS)],
                              sem).wait()

    def nll_issue(g, b):
        sem = nsem0 if b == 0 else nsem1
        iv = idxn_v[g, :]
        tg = tgtn_v[g, :]
        fi = iv * 8 + lax.shift_right_logical(tg, 7)
        pltpu.async_copy(table8.at[fi], vrow2.at[b], sem)

    def nll_wait(b):
        sem = nsem0 if b == 0 else nsem1
        pltpu.make_async_copy(table8.at[pl.ds(0, 16)], vrow2.at[b],
                              sem).wait()

    def nll_compute(g, b):
        nll_wait(b)
        iv = idxn_v[g, :]
        tg = tgtn_v[g, :]
        lane = lax.bitwise_and(tg, jnp.int32(127))
        val = plsc.load_gather(vrow2.at[b], [iota16, lane])
        lsev = plsc.load_gather(lse_v, [iv])
        acc_v[...] = acc_v[...] + (lsev - val)

    def fill_block(b):
        slab = slab0 if b == 0 else slab1
        blk = blk2.at[b]

        def tile_body(j, _):
            for g in range(8):
                cols = idx_v[j, pl.ds(16 * g, 16)]
                for s in range(8):
                    vals = plsc.load_gather(slab, [cols + (1024 * s)])
                    blk[s, pl.ds(128 * j + 16 * g, 16)] = vals
            return 0

        lax.fori_loop(0, WPT, tile_body, 0)

    slab_load(0, 0)
    nll_issue(0, 0)

    def pair_body(i, carry):
        tr0 = 2 * i
        tr1 = tr0 + 1

        slab_wait(0)
        slab_load(tr1, 1)

        @pl.when(i > 0)
        def _():
            blk_wait(0)

        @pl.when(i < NG // 2)
        def _():
            nll_issue(2 * i + 1, 1)
            nll_compute(2 * i, 0)

        fill_block(0)
        blk_out(tr0, 0)

        slab_wait(1)
        slab_load(tr1 + 1, 0)

        @pl.when(i > 0)
        def _():
            blk_wait(1)

        @pl.when(i < NG // 2 - 1)
        def _():
            nll_issue(2 * i + 2, 0)

        @pl.when(i < NG // 2)
        def _():
            nll_compute(2 * i + 1, 1)

        fill_block(1)
        blk_out(tr1, 1)
        return carry

    lax.fori_loop(0, (NTR - 1) // 2, pair_body, 0)

    slab_wait(0)
    blk_wait(0)
    fill_block(0)
    blk_out(NTR - 1, 0)
    blk_wait(1)
    blk_wait(0)

    pltpu.sync_copy(acc_v, partials.at[wid])


def kernel(idx, targets, table):
    idxf = idx.reshape(N)
    idx2 = idxf.reshape(NPT, 128)
    ct_starts = [(25 * w) // 2 for w in range(NW)]
    idxw3 = jnp.stack([idx2[s:s + WPT] for s in ct_starts])
    idxn3 = idxf.reshape(NW, NG, 16)
    tgtn3 = targets.reshape(NW, NG, 16)
    lse = _row_lse(table).reshape(VOCAB)
    tabletf = jnp.pad(table.T, ((0, 0), (0, VPAD - VOCAB))).reshape(-1)
    table_p = jnp.pad(table, ((0, 0), (0, VPAD - VOCAB)))
    table8 = table_p.reshape(VOCAB * 8, 128)
    out_t, partials = _sc_gather(tabletf, table8, idxw3, idxn3, tgtn3, lse)
    loss = _finalize(partials)[0, 0]
    return (out_t.T, loss)


# speedup vs baseline: 5.0494x; 2.2497x over previous
import functools

import jax
import jax.numpy as jnp
from jax import lax
from jax.experimental import pallas as pl
from jax.experimental.pallas import tpu as pltpu
from jax.experimental.pallas import tpu_sc as plsc

VOCAB = 1000
VPAD = 1024
B, T = 1024, 50
N = B * T
NC, NS = 2, 16
NW = NC * NS
NTR = VOCAB // 8
NPT = N // 128
WPT = 13
WCOLS = WPT * 128
PER_W = N // NW
NG = PER_W // 16


def _lse_body(table_ref, lse_ref):
    x = table_ref[...]
    m = jnp.max(x, axis=1, keepdims=True)
    s = jnp.sum(jnp.exp(x - m), axis=1, keepdims=True)
    lse_ref[...] = m + jnp.log(s)


def _row_lse(table):
    return pl.pallas_call(
        _lse_body,
        out_shape=jax.ShapeDtypeStruct((VOCAB, 1), jnp.float32),
    )(table)


def _fin_body(p_ref, loss_ref):
    loss_ref[...] = jnp.sum(p_ref[...], keepdims=True) * (1.0 / N)


def _finalize(partials):
    return pl.pallas_call(
        _fin_body,
        out_shape=jax.ShapeDtypeStruct((1, 1), jnp.float32),
    )(partials)


_mesh = plsc.VectorSubcoreMesh(core_axis_name="c", subcore_axis_name="s")


@functools.partial(
    pl.kernel,
    out_type=(
        jax.ShapeDtypeStruct((VOCAB, N), jnp.float32),
        jax.ShapeDtypeStruct((NW, 16), jnp.float32),
    ),
    mesh=_mesh,
    compiler_params=pltpu.CompilerParams(
        needs_layout_passes=False, use_tc_tiling_on_sc=True),
    scratch_types=(
        pltpu.VMEM((WPT, 128), jnp.int32),
        pltpu.VMEM((NG, 16), jnp.int32),
        pltpu.VMEM((NG, 16), jnp.int32),
        pltpu.VMEM((VOCAB,), jnp.float32),
        pltpu.VMEM((8192,), jnp.float32),
        pltpu.VMEM((8192,), jnp.float32),
        pltpu.VMEM((2, 8, WCOLS), jnp.float32),
        pltpu.VMEM((2, 16, 128), jnp.float32),
        pltpu.VMEM((16,), jnp.float32),
        pltpu.SemaphoreType.DMA,
        pltpu.SemaphoreType.DMA,
        pltpu.SemaphoreType.DMA,
        pltpu.SemaphoreType.DMA,
        pltpu.SemaphoreType.DMA,
        pltpu.SemaphoreType.DMA,
    ),
)
def _sc_gather(tabletf, table8, idxw3, idxn3, tgtn3, lse, out, partials,
               idx_v, idxn_v, tgtn_v, lse_v, slab0, slab1, blk2, vrow2,
               acc_v, ssem0, ssem1, osem0, osem1, nsem0, nsem1):
    wid = lax.axis_index("s") * NC + lax.axis_index("c")
    ct_lo = (25 * wid) // 2
    col0 = ct_lo * 128
    pltpu.sync_copy(idxw3.at[wid], idx_v)
    pltpu.sync_copy(idxn3.at[wid], idxn_v)
    pltpu.sync_copy(tgtn3.at[wid], tgtn_v)
    pltpu.sync_copy(lse, lse_v)
    acc_v[...] = jnp.zeros((16,), jnp.float32)

    iota16 = jnp.arange(16, dtype=jnp.int32)

    def slab_load(tr, b):
        sem = ssem0 if b == 0 else ssem1
        slab = slab0 if b == 0 else slab1
        pltpu.async_copy(tabletf.at[pl.ds(8192 * tr, 8192)], slab, sem)

    def slab_wait(b):
        sem = ssem0 if b == 0 else ssem1
        slab = slab0 if b == 0 else slab1
        pltpu.make_async_copy(tabletf.at[pl.ds(0, 8192)], slab, sem).wait()

    def blk_out(tr, b):
        sem = osem0 if b == 0 else osem1
        pltpu.async_copy(blk2.at[b],
                         out.at[pl.ds(8 * tr, 8), pl.ds(col0, WCOLS)], sem)

    def blk_wait(b):
        sem = osem0 if b == 0 else osem1
        pltpu.make_async_copy(blk2.at[b],
                              out.at[pl.ds(0, 8), pl.ds(col0, WCOLS)],
                              sem).wait()

    def nll_issue(g, b):
        sem = nsem0 if b == 0 else nsem1
        iv = idxn_v[g, :]
        tg = tgtn_v[g, :]
        fi = iv * 8 + lax.shift_right_logical(tg, 7)
        pltpu.async_copy(table8.at[fi], vrow2.at[b], sem)

    def nll_wait(b):
        sem = nsem0 if b == 0 else nsem1
        pltpu.make_async_copy(table8.at[pl.ds(0, 16)], vrow2.at[b],
                              sem).wait()

    def nll_compute(g, b):
        nll_wait(b)
        iv = idxn_v[g, :]
        tg = tgtn_v[g, :]
        lane = lax.bitwise_and(tg, jnp.int32(127))
        val = plsc.load_gather(vrow2.at[b], [iota16, lane])
        lsev = plsc.load_gather(lse_v, [iv])
        acc_v[...] = acc_v[...] + (lsev - val)

    def fill_block(b):
        slab = slab0 if b == 0 else slab1
        blk = blk2.at[b]

        @plsc.parallel_loop(0, WPT * 8, unroll=4)
        def tile_body(k):
            j = k // 8
            g = k % 8
            cols = idx_v[j, pl.ds(16 * g, 16)]
            for s in range(8):
                vals = plsc.load_gather(slab, [cols + (1024 * s)])
                blk[s, pl.ds(128 * j + 16 * g, 16)] = vals

    slab_load(0, 0)
    nll_issue(0, 0)

    def pair_body(i, carry):
        tr0 = 2 * i
        tr1 = tr0 + 1

        slab_wait(0)
        slab_load(tr1, 1)

        @pl.when(i > 0)
        def _():
            blk_wait(0)

        @pl.when(i < NG // 2)
        def _():
            nll_issue(2 * i + 1, 1)
            nll_compute(2 * i, 0)

        fill_block(0)
        blk_out(tr0, 0)

        slab_wait(1)
        slab_load(tr1 + 1, 0)

        @pl.when(i > 0)
        def _():
            blk_wait(1)

        @pl.when(i < NG // 2 - 1)
        def _():
            nll_issue(2 * i + 2, 0)

        @pl.when(i < NG // 2)
        def _():
            nll_compute(2 * i + 1, 1)

        fill_block(1)
        blk_out(tr1, 1)
        return carry

    lax.fori_loop(0, (NTR - 1) // 2, pair_body, 0)

    slab_wait(0)
    blk_wait(0)
    fill_block(0)
    blk_out(NTR - 1, 0)
    blk_wait(1)
    blk_wait(0)

    pltpu.sync_copy(acc_v, partials.at[wid])


def kernel(idx, targets, table):
    idxf = idx.reshape(N)
    idx2 = idxf.reshape(NPT, 128)
    ct_starts = [(25 * w) // 2 for w in range(NW)]
    idxw3 = jnp.stack([idx2[s:s + WPT] for s in ct_starts])
    idxn3 = idxf.reshape(NW, NG, 16)
    tgtn3 = targets.reshape(NW, NG, 16)
    lse = _row_lse(table).reshape(VOCAB)
    tabletf = jnp.pad(table.T, ((0, 0), (0, VPAD - VOCAB))).reshape(-1)
    table_p = jnp.pad(table, ((0, 0), (0, VPAD - VOCAB)))
    table8 = table_p.reshape(VOCAB * 8, 128)
    out_t, partials = _sc_gather(tabletf, table8, idxw3, idxn3, tgtn3, lse)
    loss = _finalize(partials)[0, 0]
    return (out_t.T, loss)


# speedup vs baseline: 5.0618x; 1.0024x over previous
import functools

import jax
import jax.numpy as jnp
from jax import lax
from jax.experimental import pallas as pl
from jax.experimental.pallas import tpu as pltpu
from jax.experimental.pallas import tpu_sc as plsc

VOCAB = 1000
VPAD = 1024
B, T = 1024, 50
N = B * T
NC, NS = 2, 16
NW = NC * NS
NTR = VOCAB // 8
NPT = N // 128
WPT = 13
WCOLS = WPT * 128
PER_W = N // NW
NG = PER_W // 16


def _lse_body(table_ref, lse_ref):
    x = table_ref[...]
    m = jnp.max(x, axis=1, keepdims=True)
    s = jnp.sum(jnp.exp(x - m), axis=1, keepdims=True)
    lse_ref[...] = m + jnp.log(s)


def _row_lse(table):
    return pl.pallas_call(
        _lse_body,
        out_shape=jax.ShapeDtypeStruct((VOCAB, 1), jnp.float32),
    )(table)


def _fin_body(p_ref, loss_ref):
    loss_ref[...] = jnp.sum(p_ref[...], keepdims=True) * (1.0 / N)


def _finalize(partials):
    return pl.pallas_call(
        _fin_body,
        out_shape=jax.ShapeDtypeStruct((1, 1), jnp.float32),
    )(partials)


_mesh = plsc.VectorSubcoreMesh(core_axis_name="c", subcore_axis_name="s")


@functools.partial(
    pl.kernel,
    out_type=(
        jax.ShapeDtypeStruct((VOCAB, N), jnp.float32),
        jax.ShapeDtypeStruct((NW, 16), jnp.float32),
    ),
    mesh=_mesh,
    compiler_params=pltpu.CompilerParams(
        needs_layout_passes=False, use_tc_tiling_on_sc=True),
    scratch_types=(
        pltpu.VMEM((WPT, 128), jnp.int32),
        pltpu.VMEM((NG, 16), jnp.int32),
        pltpu.VMEM((NG, 16), jnp.int32),
        pltpu.VMEM((VOCAB,), jnp.float32),
        pltpu.VMEM((8192,), jnp.float32),
        pltpu.VMEM((8192,), jnp.float32),
        pltpu.VMEM((2, 8, WCOLS), jnp.float32),
        pltpu.VMEM((2, 16, 128), jnp.float32),
        pltpu.VMEM((16,), jnp.float32),
        pltpu.SemaphoreType.DMA,
        pltpu.SemaphoreType.DMA,
        pltpu.SemaphoreType.DMA,
        pltpu.SemaphoreType.DMA,
        pltpu.SemaphoreType.DMA,
        pltpu.SemaphoreType.DMA,
    ),
)
def _sc_gather(tabletf, table8, idxw3, idxn3, tgtn3, lse, out, partials,
               idx_v, idxn_v, tgtn_v, lse_v, slab0, slab1, blk2, vrow2,
               acc_v, ssem0, ssem1, osem0, osem1, nsem0, nsem1):
    wid = lax.axis_index("s") * NC + lax.axis_index("c")
    ct_lo = (25 * wid) // 2
    col0 = ct_lo * 128
    pltpu.sync_copy(idxw3.at[wid], idx_v)
    pltpu.sync_copy(idxn3.at[wid], idxn_v)
    pltpu.sync_copy(tgtn3.at[wid], tgtn_v)
    pltpu.sync_copy(lse, lse_v)
    acc_v[...] = jnp.zeros((16,), jnp.float32)

    iota16 = jnp.arange(16, dtype=jnp.int32)

    def slab_load(tr, b):
        sem = ssem0 if b == 0 else ssem1
        slab = slab0 if b == 0 else slab1
        pltpu.async_copy(tabletf.at[pl.ds(8192 * tr, 8192)], slab, sem)

    def slab_wait(b):
        sem = ssem0 if b == 0 else ssem1
        slab = slab0 if b == 0 else slab1
        pltpu.make_async_copy(tabletf.at[pl.ds(0, 8192)], slab, sem).wait()

    def blk_out(tr, b):
        sem = osem0 if b == 0 else osem1
        pltpu.async_copy(blk2.at[b],
                         out.at[pl.ds(8 * tr, 8), pl.ds(col0, WCOLS)], sem)

    def blk_wait(b):
        sem = osem0 if b == 0 else osem1
        pltpu.make_async_copy(blk2.at[b],
                              out.at[pl.ds(0, 8), pl.ds(col0, WCOLS)],
                              sem).wait()

    def nll_issue(g, b):
        sem = nsem0 if b == 0 else nsem1
        iv = idxn_v[g, :]
        tg = tgtn_v[g, :]
        fi = iv * 8 + lax.shift_right_logical(tg, 7)
        pltpu.async_copy(table8.at[fi], vrow2.at[b], sem)

    def nll_wait(b):
        sem = nsem0 if b == 0 else nsem1
        pltpu.make_async_copy(table8.at[pl.ds(0, 16)], vrow2.at[b],
                              sem).wait()

    def nll_compute(g, b):
        nll_wait(b)
        iv = idxn_v[g, :]
        tg = tgtn_v[g, :]
        lane = lax.bitwise_and(tg, jnp.int32(127))
        val = plsc.load_gather(vrow2.at[b], [iota16, lane])
        lsev = plsc.load_gather(lse_v, [iv])
        acc_v[...] = acc_v[...] + (lsev - val)

    def fill_block(b):
        slab = slab0 if b == 0 else slab1
        blk = blk2.at[b]

        @plsc.parallel_loop(0, WPT * 8, unroll=8)
        def tile_body(k):
            j = k // 8
            g = k % 8
            cols = idx_v[j, pl.ds(16 * g, 16)]
            for s in range(8):
                vals = plsc.load_gather(slab, [cols + (1024 * s)])
                blk[s, pl.ds(128 * j + 16 * g, 16)] = vals

    slab_load(0, 0)
    nll_issue(0, 0)

    def pair_body(i, carry):
        tr0 = 2 * i
        tr1 = tr0 + 1

        slab_wait(0)
        slab_load(tr1, 1)

        @pl.when(i > 0)
        def _():
            blk_wait(0)

        @pl.when(i < NG // 2)
        def _():
            nll_issue(2 * i + 1, 1)
            nll_compute(2 * i, 0)

        fill_block(0)
        blk_out(tr0, 0)

        slab_wait(1)
        slab_load(tr1 + 1, 0)

        @pl.when(i > 0)
        def _():
            blk_wait(1)

        @pl.when(i < NG // 2 - 1)
        def _():
            nll_issue(2 * i + 2, 0)

        @pl.when(i < NG // 2)
        def _():
            nll_compute(2 * i + 1, 1)

        fill_block(1)
        blk_out(tr1, 1)
        return carry

    lax.fori_loop(0, (NTR - 1) // 2, pair_body, 0)

    slab_wait(0)
    blk_wait(0)
    fill_block(0)
    blk_out(NTR - 1, 0)
    blk_wait(1)
    blk_wait(0)

    pltpu.sync_copy(acc_v, partials.at[wid])


def kernel(idx, targets, table):
    idxf = idx.reshape(N)
    idx2 = idxf.reshape(NPT, 128)
    ct_starts = [(25 * w) // 2 for w in range(NW)]
    idxw3 = jnp.stack([idx2[s:s + WPT] for s in ct_starts])
    idxn3 = idxf.reshape(NW, NG, 16)
    tgtn3 = targets.reshape(NW, NG, 16)
    lse = _row_lse(table).reshape(VOCAB)
    tabletf = jnp.pad(table.T, ((0, 0), (0, VPAD - VOCAB))).reshape(-1)
    table_p = jnp.pad(table, ((0, 0), (0, VPAD - VOCAB)))
    table8 = table_p.reshape(VOCAB * 8, 128)
    out_t, partials = _sc_gather(tabletf, table8, idxw3, idxn3, tgtn3, lse)
    loss = _finalize(partials)[0, 0]
    return (out_t.T, loss)


# speedup vs baseline: 5.0650x; 1.0006x over previous
import functools

import jax
import jax.numpy as jnp
from jax import lax
from jax.experimental import pallas as pl
from jax.experimental.pallas import tpu as pltpu
from jax.experimental.pallas import tpu_sc as plsc

VOCAB = 1000
VPAD = 1024
B, T = 1024, 50
N = B * T
NC, NS = 2, 16
NW = NC * NS
NTR = VOCAB // 8
NPT = N // 128
WPT = 13
WCOLS = WPT * 128
PER_W = N // NW
NG = PER_W // 16


def _lse_body(table_ref, lse_ref):
    x = table_ref[...]
    m = jnp.max(x, axis=1, keepdims=True)
    s = jnp.sum(jnp.exp(x - m), axis=1, keepdims=True)
    lse_ref[...] = m + jnp.log(s)


def _row_lse(table):
    return pl.pallas_call(
        _lse_body,
        out_shape=jax.ShapeDtypeStruct((VOCAB, 1), jnp.float32),
    )(table)


def _fin_body(p_ref, loss_ref):
    loss_ref[...] = jnp.sum(p_ref[...], keepdims=True) * (1.0 / N)


def _finalize(partials):
    return pl.pallas_call(
        _fin_body,
        out_shape=jax.ShapeDtypeStruct((1, 1), jnp.float32),
    )(partials)


_mesh = plsc.VectorSubcoreMesh(core_axis_name="c", subcore_axis_name="s")


@functools.partial(
    pl.kernel,
    out_type=(
        jax.ShapeDtypeStruct((VOCAB, N), jnp.float32),
        jax.ShapeDtypeStruct((NW, 16), jnp.float32),
    ),
    mesh=_mesh,
    compiler_params=pltpu.CompilerParams(
        needs_layout_passes=False, use_tc_tiling_on_sc=True),
    scratch_types=(
        pltpu.VMEM((WPT, 128), jnp.int32),
        pltpu.VMEM((NG, 16), jnp.int32),
        pltpu.VMEM((NG, 16), jnp.int32),
        pltpu.VMEM((VOCAB,), jnp.float32),
        pltpu.VMEM((8000,), jnp.float32),
        pltpu.VMEM((8000,), jnp.float32),
        pltpu.VMEM((2, 8, WCOLS), jnp.float32),
        pltpu.VMEM((2, 16, 128), jnp.float32),
        pltpu.VMEM((16,), jnp.float32),
        pltpu.SemaphoreType.DMA,
        pltpu.SemaphoreType.DMA,
        pltpu.SemaphoreType.DMA,
        pltpu.SemaphoreType.DMA,
        pltpu.SemaphoreType.DMA,
        pltpu.SemaphoreType.DMA,
    ),
)
def _sc_gather(tabletf, table8, idxw3, idxn3, tgtn3, lse, out, partials,
               idx_v, idxn_v, tgtn_v, lse_v, slab0, slab1, blk2, vrow2,
               acc_v, ssem0, ssem1, osem0, osem1, nsem0, nsem1):
    wid = lax.axis_index("s") * NC + lax.axis_index("c")
    ct_lo = (25 * wid) // 2
    col0 = ct_lo * 128
    pltpu.sync_copy(idxw3.at[wid], idx_v)
    pltpu.sync_copy(idxn3.at[wid], idxn_v)
    pltpu.sync_copy(tgtn3.at[wid], tgtn_v)
    pltpu.sync_copy(lse, lse_v)
    acc_v[...] = jnp.zeros((16,), jnp.float32)

    iota16 = jnp.arange(16, dtype=jnp.int32)

    def slab_load(tr, b):
        sem = ssem0 if b == 0 else ssem1
        slab = slab0 if b == 0 else slab1
        pltpu.async_copy(tabletf.at[pl.ds(8000 * tr, 8000)], slab, sem)

    def slab_wait(b):
        sem = ssem0 if b == 0 else ssem1
        slab = slab0 if b == 0 else slab1
        pltpu.make_async_copy(tabletf.at[pl.ds(0, 8000)], slab, sem).wait()

    def blk_out(tr, b):
        sem = osem0 if b == 0 else osem1
        pltpu.async_copy(blk2.at[b],
                         out.at[pl.ds(8 * tr, 8), pl.ds(col0, WCOLS)], sem)

    def blk_wait(b):
        sem = osem0 if b == 0 else osem1
        pltpu.make_async_copy(blk2.at[b],
                              out.at[pl.ds(0, 8), pl.ds(col0, WCOLS)],
                              sem).wait()

    def nll_issue(g, b):
        sem = nsem0 if b == 0 else nsem1
        iv = idxn_v[g, :]
        tg = tgtn_v[g, :]
        fi = iv * 8 + lax.shift_right_logical(tg, 7)
        pltpu.async_copy(table8.at[fi], vrow2.at[b], sem)

    def nll_wait(b):
        sem = nsem0 if b == 0 else nsem1
        pltpu.make_async_copy(table8.at[pl.ds(0, 16)], vrow2.at[b],
                              sem).wait()

    def nll_compute(g, b):
        nll_wait(b)
        iv = idxn_v[g, :]
        tg = tgtn_v[g, :]
        lane = lax.bitwise_and(tg, jnp.int32(127))
        val = plsc.load_gather(vrow2.at[b], [iota16, lane])
        lsev = plsc.load_gather(lse_v, [iv])
        acc_v[...] = acc_v[...] + (lsev - val)

    def fill_block(b):
        slab = slab0 if b == 0 else slab1
        blk = blk2.at[b]

        @plsc.parallel_loop(0, WPT * 8, unroll=8)
        def tile_body(k):
            j = k // 8
            g = k % 8
            cols = idx_v[j, pl.ds(16 * g, 16)]
            for s in range(8):
                vals = plsc.load_gather(slab, [cols + (1000 * s)])
                blk[s, pl.ds(128 * j + 16 * g, 16)] = vals

    slab_load(0, 0)
    nll_issue(0, 0)

    def pair_body(i, carry):
        tr0 = 2 * i
        tr1 = tr0 + 1

        slab_wait(0)
        slab_load(tr1, 1)

        @pl.when(i > 0)
        def _():
            blk_wait(0)

        @pl.when(i < NG // 2)
        def _():
            nll_issue(2 * i + 1, 1)
            nll_compute(2 * i, 0)

        fill_block(0)
        blk_out(tr0, 0)

        slab_wait(1)
        slab_load(tr1 + 1, 0)

        @pl.when(i > 0)
        def _():
            blk_wait(1)

        @pl.when(i < NG // 2 - 1)
        def _():
            nll_issue(2 * i + 2, 0)

        @pl.when(i < NG // 2)
        def _():
            nll_compute(2 * i + 1, 1)

        fill_block(1)
        blk_out(tr1, 1)
        return carry

    lax.fori_loop(0, (NTR - 1) // 2, pair_body, 0)

    slab_wait(0)
    blk_wait(0)
    fill_block(0)
    blk_out(NTR - 1, 0)
    blk_wait(1)
    blk_wait(0)

    pltpu.sync_copy(acc_v, partials.at[wid])


def kernel(idx, targets, table):
    idxf = idx.reshape(N)
    idx2 = idxf.reshape(NPT, 128)
    ct_starts = [(25 * w) // 2 for w in range(NW)]
    idxw3 = jnp.stack([idx2[s:s + WPT] for s in ct_starts])
    idxn3 = idxf.reshape(NW, NG, 16)
    tgtn3 = targets.reshape(NW, NG, 16)
    lse = _row_lse(table).reshape(VOCAB)
    tabletf = table.T.reshape(-1)
    table_p = jnp.pad(table, ((0, 0), (0, VPAD - VOCAB)))
    table8 = table_p.reshape(VOCAB * 8, 128)
    out_t, partials = _sc_gather(tabletf, table8, idxw3, idxn3, tgtn3, lse)
    loss = _finalize(partials)[0, 0]
    return (out_t.T, loss)
